# baseline (speedup 1.0000x reference)
import functools

import jax
import jax.numpy as jnp
from jax import lax
from jax.experimental import pallas as pl
from jax.experimental.pallas import tpu as pltpu
from jax.experimental.pallas import tpu_sc as plsc

_VOCAB = 100000
_DIM = 16
_BATCH = 16384
_HIST = 200

_NC = 2
_NS = 16
_NW = _NC * _NS
_B_PER_W = _BATCH // _NW
_CHUNK = 16
_N_CHUNKS = _B_PER_W // _CHUNK
_UNROLL = 8
_T_COLS = _VOCAB // 8


def _proj_body(table_ref, w_ref, b_ref, out_ref):
    w128 = jnp.tile(w_ref[...], (1, 8))
    prod = table_ref[...] * w128
    c = lax.broadcasted_iota(jnp.int32, (128, 8), 0)
    j = lax.broadcasted_iota(jnp.int32, (128, 8), 1)
    sel = jnp.where(c // 16 == j, 1.0, 0.0)
    out_ref[...] = (
        lax.dot_general(
            prod, sel,
            dimension_numbers=(((1,), (0,)), ((), ())),
            preferred_element_type=jnp.float32,
        )
        + b_ref[...]
    )


def _project(table, W, b):
    return pl.pallas_call(
        _proj_body,
        out_shape=jax.ShapeDtypeStruct((_T_COLS, 8), jnp.float32),
    )(table.reshape(_T_COLS, 128), W, b.reshape(1, 1))


@functools.partial(
    pl.kernel,
    out_type=jax.ShapeDtypeStruct((_BATCH,), jnp.float32),
    mesh=plsc.VectorSubcoreMesh(core_axis_name="c", subcore_axis_name="s"),
    compiler_params=pltpu.CompilerParams(
        needs_layout_passes=False, use_tc_tiling_on_sc=False
    ),
    scratch_types=[
        pltpu.VMEM((_VOCAB,), jnp.float32),
        pltpu.VMEM((2, _HIST, _CHUNK), jnp.int32),
        pltpu.VMEM((_B_PER_W,), jnp.float32),
        pltpu.SemaphoreType.DMA,
        pltpu.SemaphoreType.DMA,
        pltpu.SemaphoreType.DMA,
    ],
)
def _sc_pool(t_hbm, x_hbm, out_hbm, t_v, x_v, out_v, sem0, sem1, sem_t):
    wid = lax.axis_index("s") * _NC + lax.axis_index("c")

    t_dma = pltpu.async_copy(t_hbm, t_v, sem_t)

    sems = (sem0, sem1)

    def start_fetch(c):
        return pltpu.async_copy(x_hbm.at[wid, c], x_v.at[c % 2], sems[c % 2])

    zero = jnp.zeros((16,), jnp.float32)

    dmas = [start_fetch(0), None]
    t_dma.wait()

    for c in range(_N_CHUNKS):
        if c + 1 < _N_CHUNKS:
            dmas[(c + 1) % 2] = start_fetch(c + 1)
        dmas[c % 2].wait()
        xc = x_v.at[c % 2]

        def inner(i, carry, xc=xc):
            a0, a1, a2, a3 = carry
            l = i * _UNROLL
            for u in range(_UNROLL):
                idx = xc[l + u, :]
                val = plsc.load_gather(t_v, [idx])
                if u % 4 == 0:
                    a0 = a0 + val
                elif u % 4 == 1:
                    a1 = a1 + val
                elif u % 4 == 2:
                    a2 = a2 + val
                else:
                    a3 = a3 + val
            return a0, a1, a2, a3

        a0, a1, a2, a3 = lax.fori_loop(
            0, _HIST // _UNROLL, inner, (zero, zero, zero, zero)
        )
        z = ((a0 + a1) + (a2 + a3)) * (1.0 / _HIST)
        out_v[pl.ds(c * _CHUNK, _CHUNK)] = 1.0 / (1.0 + jnp.exp(-z))

    pltpu.sync_copy(out_v, out_hbm.at[pl.ds(wid * _B_PER_W, _B_PER_W)])


def kernel(x, table, W, b):
    t = _project(table, W, b).reshape(_VOCAB)
    x4 = x.reshape(_NW, _N_CHUNKS, _CHUNK, _HIST).transpose(0, 1, 3, 2)
    out = _sc_pool(t, x4)
    return out.reshape(_BATCH, 1)


# speedup vs baseline: 1.4344x; 1.4344x over previous
import functools

import jax
import jax.numpy as jnp
from jax import lax
from jax.experimental import pallas as pl
from jax.experimental.pallas import tpu as pltpu
from jax.experimental.pallas import tpu_sc as plsc

_VOCAB = 100000
_DIM = 16
_BATCH = 16384
_HIST = 200

_NC = 2
_NS = 16
_NW = _NC * _NS
_B_PER_W = _BATCH // _NW
_CHUNK = 16
_N_CHUNKS = _B_PER_W // _CHUNK
_UNROLL = 8
_T_COLS = _VOCAB // 8
_T_PITCH = 12544
_OUT_ROWS_PER_W = _B_PER_W // 128


def _proj_body(table_ref, w_ref, b_ref, out_ref):
    w128 = jnp.tile(w_ref[...], (1, 8))
    prod = table_ref[...] * w128
    c = lax.broadcasted_iota(jnp.int32, (128, 8), 0)
    j = lax.broadcasted_iota(jnp.int32, (128, 8), 1)
    sel = jnp.where(c // 16 == j, 1.0, 0.0)
    out_ref[...] = (
        lax.dot_general(
            sel, prod,
            dimension_numbers=(((0,), (1,)), ((), ())),
            preferred_element_type=jnp.float32,
        )
        + b_ref[...]
    )


def _project(table, W, b):
    return pl.pallas_call(
        _proj_body,
        out_shape=jax.ShapeDtypeStruct((8, _T_COLS), jnp.float32),
    )(table.reshape(_T_COLS, 128), W, b.reshape(1, 1))


@functools.partial(
    pl.kernel,
    out_type=jax.ShapeDtypeStruct((128, 128), jnp.float32),
    mesh=plsc.VectorSubcoreMesh(core_axis_name="c", subcore_axis_name="s"),
    compiler_params=pltpu.CompilerParams(
        needs_layout_passes=False, use_tc_tiling_on_sc=False
    ),
    scratch_types=[
        pltpu.VMEM((8, _T_PITCH), jnp.float32),
        pltpu.VMEM((2, _CHUNK, _HIST), jnp.int32),
        pltpu.VMEM((_OUT_ROWS_PER_W, 128), jnp.float32),
        pltpu.SemaphoreType.DMA,
        pltpu.SemaphoreType.DMA,
        pltpu.SemaphoreType.DMA,
    ],
)
def _sc_pool(t_hbm, x_hbm, out_hbm, t_v, x_v, out_v, sem0, sem1, sem_t):
    wid = lax.axis_index("s") * _NC + lax.axis_index("c")
    row0 = wid * _B_PER_W

    t_dma = pltpu.async_copy(t_hbm, t_v, sem_t)

    sems = (sem0, sem1)

    def start_fetch(c):
        return pltpu.async_copy(
            x_hbm.at[pl.ds(row0 + c * _CHUNK, _CHUNK), :],
            x_v.at[c % 2],
            sems[c % 2],
        )

    lane = lax.iota(jnp.int32, 16)
    zero = jnp.zeros((16,), jnp.float32)
    izero = jnp.zeros((16,), jnp.int32)

    dmas = [start_fetch(0), None]
    t_dma.wait()

    for c in range(_N_CHUNKS):
        if c + 1 < _N_CHUNKS:
            dmas[(c + 1) % 2] = start_fetch(c + 1)
        dmas[c % 2].wait()
        xc = x_v.at[c % 2]

        def inner(i, carry, xc=xc):
            a0, a1, a2, a3, l = carry
            for u in range(_UNROLL):
                idx = plsc.load_gather(xc, [lane, l + u])
                val = plsc.load_gather(
                    t_v, [lax.bitwise_and(idx, 7), lax.shift_right_logical(idx, 3)]
                )
                if u % 4 == 0:
                    a0 = a0 + val
                elif u % 4 == 1:
                    a1 = a1 + val
                elif u % 4 == 2:
                    a2 = a2 + val
                else:
                    a3 = a3 + val
            return a0, a1, a2, a3, l + _UNROLL

        a0, a1, a2, a3, _ = lax.fori_loop(
            0, _HIST // _UNROLL, inner, (zero, zero, zero, zero, izero)
        )
        z = ((a0 + a1) + (a2 + a3)) * (1.0 / _HIST)
        out_v[c // 8, pl.ds((c % 8) * _CHUNK, _CHUNK)] = 1.0 / (1.0 + jnp.exp(-z))

    pltpu.sync_copy(out_v, out_hbm.at[pl.ds(wid * _OUT_ROWS_PER_W, _OUT_ROWS_PER_W), :])


def kernel(x, table, W, b):
    t = _project(table, W, b)
    t = jnp.pad(t, ((0, 0), (0, _T_PITCH - _T_COLS)))
    out = _sc_pool(t, x)
    return out.reshape(_BATCH, 1)
